# baseline (speedup 1.0000x reference)
import functools

import jax
import jax.numpy as jnp
from jax import lax
from jax.experimental import pallas as pl
from jax.experimental.pallas import tpu as pltpu
from jax.experimental.pallas import tpu_sc as plsc

D = 64
DP = 128
B = 16384
NSEC = 20
NC = 2
NS = 16
NW = NC * NS
BPW = B // NW
CH = 128
NCH = BPW // CH
LANES = 16

_mesh = plsc.VectorSubcoreMesh(core_axis_name="c", subcore_axis_name="s")


@functools.partial(
    pl.kernel,
    mesh=_mesh,
    out_type=jax.ShapeDtypeStruct((NW, BPW, D), jnp.float32),
    scratch_types=[
        pltpu.VMEM((BPW,), jnp.int32),
        pltpu.VMEM((BPW,), jnp.int32),
        pltpu.VMEM((BPW,), jnp.int32),
        pltpu.VMEM((NSEC, D), jnp.float32),
        pltpu.VMEM((BPW, DP), jnp.float32),
        pltpu.VMEM((BPW, D), jnp.float32),
        pltpu.SemaphoreType.DMA,
        [pltpu.SemaphoreType.DMA] * NCH,
        pltpu.SemaphoreType.DMA,
    ],
    compiler_params=pltpu.CompilerParams(use_tc_tiling_on_sc=False),
)
def _emb_kernel(sids_hbm, secs_hbm, stock_hbm, sector_hbm, out_hbm,
                sidx, parv, cidv, secT, bufp, buf, isem, gsems, wsem):
    wid = lax.axis_index("s") * NC + lax.axis_index("c")
    base = wid * BPW

    c1 = pltpu.async_copy(sids_hbm.at[pl.ds(base, BPW)], sidx, isem)
    c2 = pltpu.async_copy(secs_hbm.at[pl.ds(base, BPW)], cidv, isem)
    c3 = pltpu.async_copy(sector_hbm, secT, isem)
    c1.wait()
    c2.wait()
    c3.wait()

    def split(g, carry):
        sl = pl.ds(g * LANES, LANES)
        v = sidx[sl]
        parv[sl] = v & 1
        sidx[sl] = v >> 1
        return carry

    lax.fori_loop(0, BPW // LANES, split, 0)

    gathers = [
        pltpu.async_copy(
            stock_hbm.at[sidx.at[pl.ds(j * CH, CH)]],
            bufp.at[pl.ds(j * CH, CH)], gsems[j])
        for j in range(NCH)
    ]

    writes = []
    for j in range(NCH):
        gathers[j].wait()

        def body(g, carry):
            rbase = g * LANES
            sidv = cidv[pl.ds(rbase, LANES)]
            pv = parv[pl.ds(rbase, LANES)]
            for l in range(LANES):
                s = sidv[l]
                off = pv[l] * D
                r = rbase + l
                for c in range(D // LANES):
                    sl = pl.ds(c * LANES, LANES)
                    buf[r, sl] = (
                        bufp[r, pl.ds(off + c * LANES, LANES)] + secT[s, sl])
            return carry

        lax.fori_loop(j * (CH // LANES), (j + 1) * (CH // LANES), body, 0)
        writes.append(pltpu.async_copy(
            buf.at[pl.ds(j * CH, CH)], out_hbm.at[wid].at[pl.ds(j * CH, CH)],
            wsem))
    for w in writes:
        w.wait()


def kernel(stock_ids, sector_ids, stock_table, sector_table):
    paired = stock_table.reshape(50000, DP)
    return _emb_kernel(
        stock_ids, sector_ids, paired, sector_table).reshape(B, D)


# speedup vs baseline: 1.1123x; 1.1123x over previous
import functools

import jax
import jax.numpy as jnp
from jax import lax
from jax.experimental import pallas as pl
from jax.experimental.pallas import tpu as pltpu
from jax.experimental.pallas import tpu_sc as plsc

D = 64
B = 16384
NSEC = 20
NC = 2
NS = 16
NW = NC * NS
BPW = B // NW
CH = 128
NCH = BPW // CH
LANES = 16

_mesh = plsc.VectorSubcoreMesh(core_axis_name="c", subcore_axis_name="s")


@functools.partial(
    pl.kernel,
    mesh=_mesh,
    out_type=jax.ShapeDtypeStruct((NW, BPW, D), jnp.float32),
    scratch_types=[
        pltpu.VMEM((BPW,), jnp.int32),
        pltpu.VMEM((NSEC, D), jnp.float32),
        pltpu.VMEM((BPW, D), jnp.float32),
        pltpu.SemaphoreType.DMA,
        pltpu.SemaphoreType.DMA,
    ],
    compiler_params=pltpu.CompilerParams(use_tc_tiling_on_sc=False),
)
def _sector_kernel(secs_hbm, sector_hbm, out_hbm, cidv, secT, buf, isem, wsem):
    wid = lax.axis_index("s") * NC + lax.axis_index("c")
    base = wid * BPW

    c1 = pltpu.async_copy(secs_hbm.at[pl.ds(base, BPW)], cidv, isem)
    c2 = pltpu.async_copy(sector_hbm, secT, isem)
    c1.wait()
    c2.wait()

    writes = []
    for j in range(NCH):
        def body(g, carry):
            rbase = g * LANES
            sidv = cidv[pl.ds(rbase, LANES)]
            for l in range(LANES):
                s = sidv[l]
                r = rbase + l
                for c in range(D // LANES):
                    sl = pl.ds(c * LANES, LANES)
                    buf[r, sl] = secT[s, sl]
            return carry

        lax.fori_loop(j * (CH // LANES), (j + 1) * (CH // LANES), body, 0)
        writes.append(pltpu.async_copy(
            buf.at[pl.ds(j * CH, CH)], out_hbm.at[wid].at[pl.ds(j * CH, CH)],
            wsem))
    for w in writes:
        w.wait()


@functools.partial(
    pl.kernel,
    mesh=_mesh,
    out_type=jax.ShapeDtypeStruct((NW, BPW, D), jnp.float32),
    scratch_types=[
        pltpu.VMEM((BPW,), jnp.int32),
        pltpu.VMEM((BPW, D), jnp.float32),
        pltpu.SemaphoreType.DMA,
        [pltpu.SemaphoreType.DMA] * NCH,
        [pltpu.SemaphoreType.DMA] * NCH,
        pltpu.SemaphoreType.DMA,
    ],
    compiler_params=pltpu.CompilerParams(use_tc_tiling_on_sc=False),
)
def _stock_kernel(sids_hbm, partial_hbm, stock_hbm, out_hbm,
                  sidx, buf, isem, psems, gsems, wsem):
    wid = lax.axis_index("s") * NC + lax.axis_index("c")
    base = wid * BPW

    i1 = pltpu.async_copy(sids_hbm.at[pl.ds(base, BPW)], sidx, isem)
    ploads = [
        pltpu.async_copy(
            partial_hbm.at[wid].at[pl.ds(j * CH, CH)],
            buf.at[pl.ds(j * CH, CH)], psems[j])
        for j in range(NCH)
    ]
    i1.wait()

    adds = []
    for j in range(NCH):
        ploads[j].wait()
        adds.append(pltpu.async_copy(
            stock_hbm.at[sidx.at[pl.ds(j * CH, CH)]],
            buf.at[pl.ds(j * CH, CH)], gsems[j], add=True))
    writes = []
    for j in range(NCH):
        adds[j].wait()
        writes.append(pltpu.async_copy(
            buf.at[pl.ds(j * CH, CH)], out_hbm.at[wid].at[pl.ds(j * CH, CH)],
            wsem))
    for w in writes:
        w.wait()


def kernel(stock_ids, sector_ids, stock_table, sector_table):
    partial = _sector_kernel(sector_ids, sector_table)
    out = _stock_kernel(stock_ids, partial, stock_table)
    return out.reshape(B, D)
